# Initial kernel scaffold; baseline (speedup 1.0000x reference)
import jax
import jax.numpy as jnp
from jax.experimental import pallas as pl


def kernel(input, feat, embed_w, keys_w, time_w, w0, b0, w1, b1, w2, b2, w3, b3, w4, b4, w5, b5, w6, b6, w7, b7, w8, b8):
    raise NotImplementedError("write your pallas kernel here")


# speedup vs baseline: 1.4803x; 1.4803x over previous
import functools

import jax
import jax.numpy as jnp
from jax import lax
from jax.experimental import pallas as pl
from jax.experimental.pallas import tpu as pltpu
from jax.experimental.pallas import tpu_sc as plsc

LEVELS = 1024
T = 2048
D = 10000
SIGNAL_MIN = -5.0
SIGNAL_MAX = 5.0
SLICES = [(0, 3), (3, 9), (9, 12), (12, 15), (15, 18), (18, 21), (21, 24),
          (24, 27), (27, 30)]

NC = 2
NS = 16
NW = NC * NS
TPW = T // NW
LANES = 16
CH = D // LANES


def _sc_accum_body(eidx_hbm, tidx_hbm, embed_hbm, keys_hbm, time_hbm,
                   out_hbm, eidx_v, tidx_v, e3, tw, kb, acc,
                   sem_e, sem_t, sem_k):
    wid = lax.axis_index("s") * NC + lax.axis_index("c")
    base = wid * TPW
    pltpu.sync_copy(eidx_hbm.at[pl.ds(base, TPW)], eidx_v)
    pltpu.sync_copy(tidx_hbm.at[pl.ds(base, TPW)], tidx_v)

    def zbody(j, carry):
        acc[0, pl.ds(j * LANES, LANES)] = jnp.zeros((LANES,), jnp.float32)
        return carry

    lax.fori_loop(0, CH, zbody, 0)

    def tbody(i, carry):
        ce = pltpu.async_copy(embed_hbm.at[eidx_v.at[i]], e3, sem_e)
        ct = pltpu.async_copy(time_hbm.at[tidx_v.at[i]], tw, sem_t)
        ck = pltpu.async_copy(keys_hbm.at[pl.ds(base + i, 1)], kb, sem_k)
        ce.wait()
        ct.wait()
        ck.wait()

        def cbody(j, c2):
            s = pl.ds(j * LANES, LANES)
            e = e3[0, s] + e3[1, s] + e3[2, s]
            acc[0, s] += e * kb[0, s] * tw[0, s]
            return c2

        lax.fori_loop(0, CH, cbody, 0)
        return carry

    lax.fori_loop(0, TPW, tbody, 0)
    pltpu.sync_copy(acc, out_hbm.at[pl.ds(wid, 1)])


@functools.lru_cache(maxsize=1)
def _get_sc_accum():
    mesh = plsc.VectorSubcoreMesh(
        core_axis_name="c", subcore_axis_name="s",
        num_cores=NC, num_subcores=NS)
    return pl.kernel(
        _sc_accum_body,
        out_type=jax.ShapeDtypeStruct((NW, D), jnp.float32),
        mesh=mesh,
        scratch_types=[
            pltpu.VMEM((TPW, 3), jnp.int32),
            pltpu.VMEM((TPW, 1), jnp.int32),
            pltpu.VMEM((3, D), jnp.float32),
            pltpu.VMEM((1, D), jnp.float32),
            pltpu.VMEM((1, D), jnp.float32),
            pltpu.VMEM((1, D), jnp.float32),
            pltpu.SemaphoreType.DMA,
            pltpu.SemaphoreType.DMA,
            pltpu.SemaphoreType.DMA,
        ],
        compiler_params=pltpu.CompilerParams(use_tc_tiling_on_sc=False),
    )


def _tc_combine_body(partial_ref, f_ref, out_ref):
    s = jnp.sum(partial_ref[...], axis=0, keepdims=True)
    v = s * f_ref[...]
    out_ref[...] = jnp.where(v > 0, 1.0, -1.0).astype(jnp.float32)


def _level_idx(x, low, high, num):
    xc = jnp.clip(x, low, high)
    return jnp.round((xc - low) / (high - low) * (num - 1)).astype(jnp.int32)


def kernel(input, feat, embed_w, keys_w, time_w, w0, b0, w1, b1, w2, b2, w3,
           b3, w4, b4, w5, b5, w6, b6, w7, b7, w8, b8):
    eidx = _level_idx(input[:, 1:], SIGNAL_MIN, SIGNAL_MAX, LEVELS)
    tidx = _level_idx(input[:, 0], 0.0, float(T), T).reshape(T, 1)

    partial = _get_sc_accum()(eidx, tidx, embed_w, keys_w, time_w)

    ws = [w0, w1, w2, w3, w4, w5, w6, w7, w8]
    bs = [b0, b1, b2, b3, b4, b5, b6, b7, b8]
    fs = []
    for i, (lo, hi) in enumerate(SLICES):
        p = feat[lo:hi] @ ws[i].T
        fs.append(jnp.cos(p + bs[i]) * jnp.sin(p))
    ftot = fs[0] * (fs[1] + fs[8]) * (fs[2] + fs[3] + fs[4]) * (
        fs[5] + fs[6] + fs[7])

    out = pl.pallas_call(
        _tc_combine_body,
        out_shape=jax.ShapeDtypeStruct((1, D), jnp.float32),
        in_specs=[
            pl.BlockSpec((NW, D), lambda: (0, 0)),
            pl.BlockSpec((1, D), lambda: (0, 0)),
        ],
        out_specs=pl.BlockSpec((1, D), lambda: (0, 0)),
    )(partial, ftot.reshape(1, D))
    return out.reshape(D)


# speedup vs baseline: 1.8510x; 1.2505x over previous
import functools

import jax
import jax.numpy as jnp
from jax import lax
from jax.experimental import pallas as pl
from jax.experimental.pallas import tpu as pltpu
from jax.experimental.pallas import tpu_sc as plsc

LEVELS = 1024
T = 2048
D = 10000
SIGNAL_MIN = -5.0
SIGNAL_MAX = 5.0
SLICES = [(0, 3), (3, 9), (9, 12), (12, 15), (15, 18), (18, 21), (21, 24),
          (24, 27), (27, 30)]

NC = 2
NS = 16
NW = NC * NS
TPW = T // NW
LANES = 16
CH = D // LANES


def _sc_accum_body(eidx_hbm, tidx_hbm, embed_hbm, keys_hbm, time_hbm,
                   out_hbm, eidx_v, tidx_v, e3, tw, kb, acc,
                   sem_e0, sem_t0, sem_k0, sem_e1, sem_t1, sem_k1):
    wid = lax.axis_index("s") * NC + lax.axis_index("c")
    base = wid * TPW
    pltpu.sync_copy(eidx_hbm.at[pl.ds(base, TPW)], eidx_v)
    pltpu.sync_copy(tidx_hbm.at[pl.ds(base, TPW)], tidx_v)

    def zbody(j, carry):
        acc[0, pl.ds(j * LANES, LANES)] = jnp.zeros((LANES,), jnp.float32)
        return carry

    lax.fori_loop(0, CH, zbody, 0)

    slots = [(e3.at[0], tw.at[0], kb.at[0], sem_e0, sem_t0, sem_k0),
             (e3.at[1], tw.at[1], kb.at[1], sem_e1, sem_t1, sem_k1)]

    def issue(t, b):
        e3s, tws, kbs, se, st, sk = slots[b]
        pltpu.async_copy(embed_hbm.at[eidx_v.at[t]], e3s, se)
        pltpu.async_copy(time_hbm.at[tidx_v.at[t]], tws, st)
        pltpu.async_copy(keys_hbm.at[pl.ds(base + t, 1)], kbs, sk)

    def wait(t, b):
        e3s, tws, kbs, se, st, sk = slots[b]
        pltpu.make_async_copy(embed_hbm.at[eidx_v.at[t]], e3s, se).wait()
        pltpu.make_async_copy(time_hbm.at[tidx_v.at[t]], tws, st).wait()
        pltpu.make_async_copy(keys_hbm.at[pl.ds(base + t, 1)], kbs, sk).wait()

    def accum(b):
        e3s, tws, kbs, _, _, _ = slots[b]

        def cbody(j, c2):
            s = pl.ds(j * LANES, LANES)
            e = e3s[0, s] + e3s[1, s] + e3s[2, s]
            acc[0, s] += e * kbs[0, s] * tws[0, s]
            return c2

        lax.fori_loop(0, CH, cbody, 0)

    issue(0, 0)
    issue(1, 1)

    def tbody(i, carry):
        t = 2 * i
        wait(t, 0)
        accum(0)
        issue(t + 2, 0)
        wait(t + 1, 1)
        accum(1)
        issue(t + 3, 1)
        return carry

    lax.fori_loop(0, TPW // 2 - 1, tbody, 0)
    wait(TPW - 2, 0)
    accum(0)
    wait(TPW - 1, 1)
    accum(1)
    pltpu.sync_copy(acc, out_hbm.at[pl.ds(wid, 1)])


@functools.lru_cache(maxsize=1)
def _get_sc_accum():
    mesh = plsc.VectorSubcoreMesh(
        core_axis_name="c", subcore_axis_name="s",
        num_cores=NC, num_subcores=NS)
    return pl.kernel(
        _sc_accum_body,
        out_type=jax.ShapeDtypeStruct((NW, D), jnp.float32),
        mesh=mesh,
        scratch_types=[
            pltpu.VMEM((TPW, 3), jnp.int32),
            pltpu.VMEM((TPW, 1), jnp.int32),
            pltpu.VMEM((2, 3, D), jnp.float32),
            pltpu.VMEM((2, 1, D), jnp.float32),
            pltpu.VMEM((2, 1, D), jnp.float32),
            pltpu.VMEM((1, D), jnp.float32),
            pltpu.SemaphoreType.DMA,
            pltpu.SemaphoreType.DMA,
            pltpu.SemaphoreType.DMA,
            pltpu.SemaphoreType.DMA,
            pltpu.SemaphoreType.DMA,
            pltpu.SemaphoreType.DMA,
        ],
        compiler_params=pltpu.CompilerParams(use_tc_tiling_on_sc=False),
    )


def _tc_combine_body(partial_ref, f_ref, out_ref):
    s = jnp.sum(partial_ref[...], axis=0, keepdims=True)
    v = s * f_ref[...]
    out_ref[...] = jnp.where(v > 0, 1.0, -1.0).astype(jnp.float32)


def _level_idx(x, low, high, num):
    xc = jnp.clip(x, low, high)
    return jnp.round((xc - low) / (high - low) * (num - 1)).astype(jnp.int32)


def kernel(input, feat, embed_w, keys_w, time_w, w0, b0, w1, b1, w2, b2, w3,
           b3, w4, b4, w5, b5, w6, b6, w7, b7, w8, b8):
    eidx = _level_idx(input[:, 1:], SIGNAL_MIN, SIGNAL_MAX, LEVELS)
    tidx = _level_idx(input[:, 0], 0.0, float(T), T).reshape(T, 1)

    partial = _get_sc_accum()(eidx, tidx, embed_w, keys_w, time_w)

    ws = [w0, w1, w2, w3, w4, w5, w6, w7, w8]
    bs = [b0, b1, b2, b3, b4, b5, b6, b7, b8]
    fs = []
    for i, (lo, hi) in enumerate(SLICES):
        p = feat[lo:hi] @ ws[i].T
        fs.append(jnp.cos(p + bs[i]) * jnp.sin(p))
    ftot = fs[0] * (fs[1] + fs[8]) * (fs[2] + fs[3] + fs[4]) * (
        fs[5] + fs[6] + fs[7])

    out = pl.pallas_call(
        _tc_combine_body,
        out_shape=jax.ShapeDtypeStruct((1, D), jnp.float32),
        in_specs=[
            pl.BlockSpec((NW, D), lambda: (0, 0)),
            pl.BlockSpec((1, D), lambda: (0, 0)),
        ],
        out_specs=pl.BlockSpec((1, D), lambda: (0, 0)),
    )(partial, ftot.reshape(1, D))
    return out.reshape(D)
